# baseline (speedup 1.0000x reference)
import jax
import jax.numpy as jnp
from jax.experimental import pallas as pl
from jax.experimental.pallas import tpu as pltpu


def _conv1d(x, w, b, stride, pad):
    y = jax.lax.conv_general_dilated(x, w, (stride,), [(pad, pad)],
                                     dimension_numbers=('NCH', 'OIH', 'NCH'))
    return y + b[None, :, None]


def _conv_t1d(x, w, b, stride, pad):
    k = w.shape[2]
    w2 = jnp.transpose(jnp.flip(w, axis=2), (1, 0, 2))
    y = jax.lax.conv_general_dilated(x, w2, (1,), [(k - 1 - pad, k - 1 - pad)],
                                     lhs_dilation=(stride,),
                                     dimension_numbers=('NCH', 'OIH', 'NCH'))
    return y + b[None, :, None]


_BLK_M = 512
_BLK_N = 2048
_BIG = 2 ** 30


def _vq_argmin_kernel(zsq_ref, cbsq_ref, z_ref, cb_ref, idx_ref,
                      best_ref, bidx_ref):
    j = pl.program_id(1)
    nj = pl.num_programs(1)

    zc = jax.lax.dot_general(z_ref[...], cb_ref[...], (((1,), (1,)), ((), ())),
                             preferred_element_type=jnp.float32)
    d = (zsq_ref[...] + cbsq_ref[...]) - 2.0 * zc
    lmin = jnp.min(d, axis=1, keepdims=True)
    col = jax.lax.broadcasted_iota(jnp.int32, d.shape, 1) + j * _BLK_N
    lidx = jnp.min(jnp.where(d == lmin, col, _BIG), axis=1, keepdims=True)

    @pl.when(j == 0)
    def _init():
        best_ref[...] = lmin
        bidx_ref[...] = lidx

    @pl.when(j > 0)
    def _combine():
        acc = best_ref[...].astype(jnp.bfloat16).astype(jnp.float32)
        acci = bidx_ref[...]
        upd = (lmin < acc) | ((lmin == acc) & (lidx < acci))
        best_ref[...] = jnp.where(upd, lmin, acc)
        bidx_ref[...] = jnp.where(upd, lidx, acci)

    @pl.when(j == nj - 1)
    def _emit():
        idx_ref[...] = bidx_ref[...]


def _vq_argmin(zf, codebook, z_sq, cb_sq):
    n, d = zf.shape
    k = codebook.shape[0]
    grid = (n // _BLK_M, k // _BLK_N)
    idx = pl.pallas_call(
        _vq_argmin_kernel,
        grid=grid,
        in_specs=[
            pl.BlockSpec((_BLK_M, 1), lambda i, j: (i, 0)),
            pl.BlockSpec((1, _BLK_N), lambda i, j: (0, j)),
            pl.BlockSpec((_BLK_M, d), lambda i, j: (i, 0)),
            pl.BlockSpec((_BLK_N, d), lambda i, j: (j, 0)),
        ],
        out_specs=pl.BlockSpec((_BLK_M, 1), lambda i, j: (i, 0)),
        out_shape=jax.ShapeDtypeStruct((n, 1), jnp.int32),
        scratch_shapes=[
            pltpu.VMEM((_BLK_M, 1), jnp.float32),
            pltpu.VMEM((_BLK_M, 1), jnp.int32),
        ],
    )(z_sq, cb_sq.reshape(1, k), zf, codebook)
    return idx[:, 0]


def kernel(x, enc_w1, enc_b1, enc_w2, enc_b2, enc_w3, enc_b3, codebook,
           dec_w1, dec_b1, dec_wt2, dec_bt2, dec_wt3, dec_bt3):
    xc = jnp.transpose(x, (0, 2, 1))
    h = jax.nn.relu(_conv1d(xc, enc_w1, enc_b1, 2, 1))
    h = h.astype(jnp.bfloat16).astype(jnp.float32)
    h = jax.nn.relu(_conv1d(h, enc_w2, enc_b2, 2, 1))
    h = h.astype(jnp.bfloat16).astype(jnp.float32)
    z = _conv1d(h, enc_w3, enc_b3, 1, 1)
    zt = jnp.transpose(z, (0, 2, 1))
    zf = zt.reshape(-1, zt.shape[-1])
    z_sq = jnp.sum(zf ** 2, axis=1, keepdims=True)
    cb_sq = jnp.sum(codebook ** 2, axis=1)
    idx = _vq_argmin(zf, codebook, z_sq, cb_sq)
    zq = jnp.take(codebook, idx, axis=0).reshape(zt.shape)
    commit_loss = jnp.mean((jax.lax.stop_gradient(zq) - zt) ** 2)
    zq_st = zt + jax.lax.stop_gradient(zq - zt)
    dq = jnp.transpose(zq_st, (0, 2, 1))
    r = jax.nn.relu(_conv1d(dq, dec_w1, dec_b1, 1, 1))
    r = jax.nn.relu(_conv_t1d(r, dec_wt2, dec_bt2, 2, 1))
    r = _conv_t1d(r, dec_wt3, dec_bt3, 2, 1)
    recon = jnp.transpose(r, (0, 2, 1))
    return (recon, zq_st, commit_loss)


# speedup vs baseline: 1.0458x; 1.0458x over previous
import jax
import jax.numpy as jnp
from jax.experimental import pallas as pl
from jax.experimental.pallas import tpu as pltpu


def _conv1d(x, w, b, stride, pad):
    y = jax.lax.conv_general_dilated(x, w, (stride,), [(pad, pad)],
                                     dimension_numbers=('NCH', 'OIH', 'NCH'))
    return y + b[None, :, None]


def _conv_t1d(x, w, b, stride, pad):
    k = w.shape[2]
    w2 = jnp.transpose(jnp.flip(w, axis=2), (1, 0, 2))
    y = jax.lax.conv_general_dilated(x, w2, (1,), [(k - 1 - pad, k - 1 - pad)],
                                     lhs_dilation=(stride,),
                                     dimension_numbers=('NCH', 'OIH', 'NCH'))
    return y + b[None, :, None]


_BLK_M = 1024
_BLK_N = 2048
_BIG = 2 ** 30


def _vq_argmin_kernel(zsq_ref, cbsq_ref, z_ref, cb_ref, idx_ref,
                      best_ref, bidx_ref):
    j = pl.program_id(1)
    nj = pl.num_programs(1)

    zc = jax.lax.dot_general(z_ref[...], cb_ref[...], (((1,), (1,)), ((), ())),
                             preferred_element_type=jnp.float32)
    d = (zsq_ref[...] + cbsq_ref[...]) - 2.0 * zc
    lmin = jnp.min(d, axis=1, keepdims=True)
    col = jax.lax.broadcasted_iota(jnp.int32, d.shape, 1) + j * _BLK_N
    lidx = jnp.min(jnp.where(d == lmin, col, _BIG), axis=1, keepdims=True)

    @pl.when(j == 0)
    def _init():
        best_ref[...] = lmin
        bidx_ref[...] = lidx

    @pl.when(j > 0)
    def _combine():
        acc = best_ref[...].astype(jnp.bfloat16).astype(jnp.float32)
        acci = bidx_ref[...]
        upd = (lmin < acc) | ((lmin == acc) & (lidx < acci))
        best_ref[...] = jnp.where(upd, lmin, acc)
        bidx_ref[...] = jnp.where(upd, lidx, acci)

    @pl.when(j == nj - 1)
    def _emit():
        idx_ref[...] = bidx_ref[...]


def _vq_argmin(zf, codebook, z_sq, cb_sq):
    n, d = zf.shape
    k = codebook.shape[0]
    grid = (n // _BLK_M, k // _BLK_N)
    idx = pl.pallas_call(
        _vq_argmin_kernel,
        grid=grid,
        in_specs=[
            pl.BlockSpec((_BLK_M, 1), lambda i, j: (i, 0)),
            pl.BlockSpec((1, _BLK_N), lambda i, j: (0, j)),
            pl.BlockSpec((_BLK_M, d), lambda i, j: (i, 0)),
            pl.BlockSpec((_BLK_N, d), lambda i, j: (j, 0)),
        ],
        out_specs=pl.BlockSpec((_BLK_M, 1), lambda i, j: (i, 0)),
        out_shape=jax.ShapeDtypeStruct((n, 1), jnp.int32),
        scratch_shapes=[
            pltpu.VMEM((_BLK_M, 1), jnp.float32),
            pltpu.VMEM((_BLK_M, 1), jnp.int32),
        ],
    )(z_sq, cb_sq.reshape(1, k), zf, codebook)
    return idx[:, 0]


def kernel(x, enc_w1, enc_b1, enc_w2, enc_b2, enc_w3, enc_b3, codebook,
           dec_w1, dec_b1, dec_wt2, dec_bt2, dec_wt3, dec_bt3):
    xc = jnp.transpose(x, (0, 2, 1))
    h = jax.nn.relu(_conv1d(xc, enc_w1, enc_b1, 2, 1))
    h = h.astype(jnp.bfloat16).astype(jnp.float32)
    h = jax.nn.relu(_conv1d(h, enc_w2, enc_b2, 2, 1))
    h = h.astype(jnp.bfloat16).astype(jnp.float32)
    z = _conv1d(h, enc_w3, enc_b3, 1, 1)
    zt = jnp.transpose(z, (0, 2, 1))
    zf = zt.reshape(-1, zt.shape[-1])
    z_sq = jnp.sum(zf ** 2, axis=1, keepdims=True)
    cb_sq = jnp.sum(codebook ** 2, axis=1)
    idx = _vq_argmin(zf, codebook, z_sq, cb_sq)
    zq = jnp.take(codebook, idx, axis=0).reshape(zt.shape)
    commit_loss = jnp.mean((jax.lax.stop_gradient(zq) - zt) ** 2)
    zq_st = zt + jax.lax.stop_gradient(zq - zt)
    dq = jnp.transpose(zq_st, (0, 2, 1))
    r = jax.nn.relu(_conv1d(dq, dec_w1, dec_b1, 1, 1))
    r = jax.nn.relu(_conv_t1d(r, dec_wt2, dec_bt2, 2, 1))
    r = _conv_t1d(r, dec_wt3, dec_bt3, 2, 1)
    recon = jnp.transpose(r, (0, 2, 1))
    return (recon, zq_st, commit_loss)


# speedup vs baseline: 1.0724x; 1.0255x over previous
import jax
import jax.numpy as jnp
from jax.experimental import pallas as pl
from jax.experimental.pallas import tpu as pltpu


def _conv1d(x, w, b, stride, pad):
    y = jax.lax.conv_general_dilated(x, w, (stride,), [(pad, pad)],
                                     dimension_numbers=('NCH', 'OIH', 'NCH'))
    return y + b[None, :, None]


def _conv_t1d(x, w, b, stride, pad):
    k = w.shape[2]
    w2 = jnp.transpose(jnp.flip(w, axis=2), (1, 0, 2))
    y = jax.lax.conv_general_dilated(x, w2, (1,), [(k - 1 - pad, k - 1 - pad)],
                                     lhs_dilation=(stride,),
                                     dimension_numbers=('NCH', 'OIH', 'NCH'))
    return y + b[None, :, None]


_BLK_M = 2048
_BLK_N = 2048
_BIG = 2 ** 30


def _vq_argmin_kernel(zsq_ref, cbsq_ref, z_ref, cb_ref, idx_ref,
                      best_ref, bidx_ref):
    j = pl.program_id(1)
    nj = pl.num_programs(1)

    zc = jax.lax.dot_general(z_ref[...], cb_ref[...], (((1,), (1,)), ((), ())),
                             preferred_element_type=jnp.float32)
    d = (zsq_ref[...] + cbsq_ref[...]) - 2.0 * zc
    lmin = jnp.min(d, axis=1, keepdims=True)
    col = jax.lax.broadcasted_iota(jnp.int32, d.shape, 1) + j * _BLK_N
    lidx = jnp.min(jnp.where(d == lmin, col, _BIG), axis=1, keepdims=True)

    @pl.when(j == 0)
    def _init():
        best_ref[...] = lmin
        bidx_ref[...] = lidx

    @pl.when(j > 0)
    def _combine():
        acc = best_ref[...].astype(jnp.bfloat16).astype(jnp.float32)
        acci = bidx_ref[...]
        upd = (lmin < acc) | ((lmin == acc) & (lidx < acci))
        best_ref[...] = jnp.where(upd, lmin, acc)
        bidx_ref[...] = jnp.where(upd, lidx, acci)

    @pl.when(j == nj - 1)
    def _emit():
        idx_ref[...] = bidx_ref[...]


def _vq_argmin(zf, codebook, z_sq, cb_sq):
    n, d = zf.shape
    k = codebook.shape[0]
    grid = (n // _BLK_M, k // _BLK_N)
    idx = pl.pallas_call(
        _vq_argmin_kernel,
        grid=grid,
        in_specs=[
            pl.BlockSpec((_BLK_M, 1), lambda i, j: (i, 0)),
            pl.BlockSpec((1, _BLK_N), lambda i, j: (0, j)),
            pl.BlockSpec((_BLK_M, d), lambda i, j: (i, 0)),
            pl.BlockSpec((_BLK_N, d), lambda i, j: (j, 0)),
        ],
        out_specs=pl.BlockSpec((_BLK_M, 1), lambda i, j: (i, 0)),
        out_shape=jax.ShapeDtypeStruct((n, 1), jnp.int32),
        scratch_shapes=[
            pltpu.VMEM((_BLK_M, 1), jnp.float32),
            pltpu.VMEM((_BLK_M, 1), jnp.int32),
        ],
    )(z_sq, cb_sq.reshape(1, k), zf, codebook)
    return idx[:, 0]


def kernel(x, enc_w1, enc_b1, enc_w2, enc_b2, enc_w3, enc_b3, codebook,
           dec_w1, dec_b1, dec_wt2, dec_bt2, dec_wt3, dec_bt3):
    xc = jnp.transpose(x, (0, 2, 1))
    h = jax.nn.relu(_conv1d(xc, enc_w1, enc_b1, 2, 1))
    h = h.astype(jnp.bfloat16).astype(jnp.float32)
    h = jax.nn.relu(_conv1d(h, enc_w2, enc_b2, 2, 1))
    h = h.astype(jnp.bfloat16).astype(jnp.float32)
    z = _conv1d(h, enc_w3, enc_b3, 1, 1)
    zt = jnp.transpose(z, (0, 2, 1))
    zf = zt.reshape(-1, zt.shape[-1])
    z_sq = jnp.sum(zf ** 2, axis=1, keepdims=True)
    cb_sq = jnp.sum(codebook ** 2, axis=1)
    idx = _vq_argmin(zf, codebook, z_sq, cb_sq)
    zq = jnp.take(codebook, idx, axis=0).reshape(zt.shape)
    commit_loss = jnp.mean((jax.lax.stop_gradient(zq) - zt) ** 2)
    zq_st = zt + jax.lax.stop_gradient(zq - zt)
    dq = jnp.transpose(zq_st, (0, 2, 1))
    r = jax.nn.relu(_conv1d(dq, dec_w1, dec_b1, 1, 1))
    r = jax.nn.relu(_conv_t1d(r, dec_wt2, dec_bt2, 2, 1))
    r = _conv_t1d(r, dec_wt3, dec_bt3, 2, 1)
    recon = jnp.transpose(r, (0, 2, 1))
    return (recon, zq_st, commit_loss)
